# Initial kernel scaffold; baseline (speedup 1.0000x reference)
import jax
import jax.numpy as jnp
from jax.experimental import pallas as pl


def kernel(x, edge_index, edge_attr, W_in, b_in, We, be, eps, W1, b1, g1, bt1, W2, b2, bn_g, bn_b, Wc, bc):
    raise NotImplementedError("write your pallas kernel here")


# speedup vs baseline: 2.4931x; 2.4931x over previous
import functools

import jax
import jax.numpy as jnp
from jax import lax
from jax.experimental import pallas as pl
from jax.experimental.pallas import tpu as pltpu
from jax.experimental.pallas import tpu_sc as plsc

_N = 10000
_E = 320000
_DIN = 128
_H = 64
_ED = 16
_L = 3
_OUT = 2
_BN_EPS = 1e-5
_PN_EPS = 1e-5

_NC = 2
_NS = 16
_TILES = _NC * _NS
_EPT = _E // _TILES
_CH = 80
_NCH = _EPT // _CH
_RPT = 624
_RPT_LAST = _N - (_NS - 1) * _RPT


def _mm_bias_body(a_ref, w_ref, b_ref, o_ref):
    o_ref[...] = (
        jnp.dot(a_ref[...], w_ref[...], preferred_element_type=jnp.float32)
        + b_ref[...]
    )


def _mm_bias(a, w, b, block_rows):
    m, k = a.shape
    _, h = w.shape
    return pl.pallas_call(
        _mm_bias_body,
        grid=(m // block_rows,),
        in_specs=[
            pl.BlockSpec((block_rows, k), lambda i: (i, 0)),
            pl.BlockSpec((k, h), lambda i: (0, 0)),
            pl.BlockSpec((1, h), lambda i: (0, 0)),
        ],
        out_specs=pl.BlockSpec((block_rows, h), lambda i: (i, 0)),
        out_shape=jax.ShapeDtypeStruct((m, h), jnp.float32),
    )(a, w, b.reshape(1, h))


def _node_update_body(h_ref, p_ref, eps_ref, w1_ref, b1_ref, g1_ref, bt1_ref,
                      w2_ref, b2_ref, bg_ref, bb_ref, o_ref, *, mode):
    bn_scale = 1.0 / jnp.sqrt(1.0 + _BN_EPS)
    z = (1.0 + eps_ref[0, 0]) * h_ref[...] + p_ref[0] + p_ref[1]
    z = jnp.dot(z, w1_ref[...], preferred_element_type=jnp.float32) + b1_ref[...]
    z = z * (g1_ref[...] * bn_scale) + bt1_ref[...]
    z = jnp.maximum(z, 0.0)
    z = jnp.dot(z, w2_ref[...], preferred_element_type=jnp.float32) + b2_ref[...]
    z = z * (bg_ref[...] * bn_scale) + bb_ref[...]
    z = jnp.maximum(z, 0.0)
    if mode == "pairnorm":
        z = z - jnp.mean(z, axis=0, keepdims=True)
        denom = jnp.sqrt(_PN_EPS + jnp.mean(jnp.sum(z * z, axis=-1)))
        z = z / denom
    o_ref[...] = z


def _final_body(h_ref, p_ref, eps_ref, w1_ref, b1_ref, g1_ref, bt1_ref,
                w2_ref, b2_ref, bg_ref, bb_ref, wc_ref, bc_ref, o_ref):
    bn_scale = 1.0 / jnp.sqrt(1.0 + _BN_EPS)
    z = (1.0 + eps_ref[0, 0]) * h_ref[...] + p_ref[0] + p_ref[1]
    z = jnp.dot(z, w1_ref[...], preferred_element_type=jnp.float32) + b1_ref[...]
    z = z * (g1_ref[...] * bn_scale) + bt1_ref[...]
    z = jnp.maximum(z, 0.0)
    z = jnp.dot(z, w2_ref[...], preferred_element_type=jnp.float32) + b2_ref[...]
    z = z * (bg_ref[...] * bn_scale) + bb_ref[...]
    z = jnp.maximum(z, 0.0)
    o_ref[...] = (
        jnp.dot(z, wc_ref[...], preferred_element_type=jnp.float32) + bc_ref[...]
    )


def _node_update(h, parts, eps_l, w1, b1, g1, bt1, w2, b2, bg, bb):
    return pl.pallas_call(
        functools.partial(_node_update_body, mode="pairnorm"),
        out_shape=jax.ShapeDtypeStruct((_N, _H), jnp.float32),
    )(h, parts, eps_l.reshape(1, 1), w1, b1.reshape(1, _H), g1.reshape(1, _H),
      bt1.reshape(1, _H), w2, b2.reshape(1, _H), bg.reshape(1, _H),
      bb.reshape(1, _H))


def _node_final(h, parts, eps_l, w1, b1, g1, bt1, w2, b2, bg, bb, wc, bc):
    return pl.pallas_call(
        _final_body,
        out_shape=jax.ShapeDtypeStruct((_N, _OUT), jnp.float32),
    )(h, parts, eps_l.reshape(1, 1), w1, b1.reshape(1, _H), g1.reshape(1, _H),
      bt1.reshape(1, _H), w2, b2.reshape(1, _H), bg.reshape(1, _H),
      bb.reshape(1, _H), wc, bc.reshape(1, _OUT))


def _sc_agg_body(src_hbm, dst_hbm, e_hbm, h_hbm, z_hbm, out_hbm,
                 src_v, dst_v, e_v, g_v, agg_sh, sem):
    c = lax.axis_index("c")
    s = lax.axis_index("s")
    wid = c * _NS + s

    row0 = pl.multiple_of(s * _RPT, 8)
    last = s == _NS - 1

    @pl.when(~last)
    def _zero_main():
        pltpu.sync_copy(z_hbm.at[pl.ds(0, _RPT)], agg_sh.at[pl.ds(row0, _RPT)])

    @pl.when(last)
    def _zero_last():
        pltpu.sync_copy(z_hbm, agg_sh.at[pl.ds((_NS - 1) * _RPT, _RPT_LAST)])

    plsc.subcore_barrier()

    base = wid * _EPT

    def chunk(i, carry):
        off = pl.multiple_of(base + i * _CH, _CH)
        pltpu.sync_copy(src_hbm.at[pl.ds(off, _CH)], src_v)
        pltpu.sync_copy(dst_hbm.at[pl.ds(off, _CH)], dst_v)
        pltpu.sync_copy(e_hbm.at[pl.ds(off, _CH)], e_v)
        pltpu.async_copy(h_hbm.at[src_v], g_v, sem).wait()

        def row(r, carry2):
            for j in range(_H // 16):
                sl = pl.ds(j * 16, 16)
                g_v[r, sl] = jnp.maximum(g_v[r, sl] + e_v[r, sl], 0.0)
            return carry2

        lax.fori_loop(0, _CH, row, 0)
        pltpu.sync_copy(g_v, agg_sh.at[dst_v], add=True)
        return carry

    lax.fori_loop(0, _NCH, chunk, 0)
    plsc.subcore_barrier()

    @pl.when(~last)
    def _read_main():
        pltpu.sync_copy(agg_sh.at[pl.ds(row0, _RPT)],
                        out_hbm.at[c, pl.ds(row0, _RPT)])

    @pl.when(last)
    def _read_last():
        pltpu.sync_copy(agg_sh.at[pl.ds((_NS - 1) * _RPT, _RPT_LAST)],
                        out_hbm.at[c, pl.ds((_NS - 1) * _RPT, _RPT_LAST)])


_sc_agg = functools.partial(
    pl.kernel,
    mesh=plsc.VectorSubcoreMesh(core_axis_name="c", subcore_axis_name="s"),
    out_type=jax.ShapeDtypeStruct((_NC, _N, _H), jnp.float32),
    scratch_types=[
        pltpu.VMEM((_CH,), jnp.int32),
        pltpu.VMEM((_CH,), jnp.int32),
        pltpu.VMEM((_CH, _H), jnp.float32),
        pltpu.VMEM((_CH, _H), jnp.float32),
        pltpu.VMEM_SHARED((_N, _H), jnp.float32),
        pltpu.SemaphoreType.DMA,
    ],
    compiler_params=pltpu.CompilerParams(use_tc_tiling_on_sc=False),
)(_sc_agg_body)


def kernel(x, edge_index, edge_attr, W_in, b_in, We, be, eps, W1, b1, g1, bt1,
           W2, b2, bn_g, bn_b, Wc, bc):
    src = edge_index[0]
    dst = edge_index[1]
    zrows = jnp.zeros((_RPT_LAST, _H), jnp.float32)

    h = _mm_bias(x, W_in, b_in, 2000)
    out = None
    for l in range(_L):
        e = _mm_bias(edge_attr, We[l], be[l], 8000)
        parts = _sc_agg(src, dst, e, h, zrows)
        if l < _L - 1:
            h = _node_update(h, parts, eps[l], W1[l], b1[l], g1[l], bt1[l],
                             W2[l], b2[l], bn_g[l], bn_b[l])
        else:
            out = _node_final(h, parts, eps[l], W1[l], b1[l], g1[l], bt1[l],
                              W2[l], b2[l], bn_g[l], bn_b[l], Wc, bc)
    return out


# speedup vs baseline: 4.0739x; 1.6341x over previous
import functools

import jax
import jax.numpy as jnp
from jax import lax
from jax.experimental import pallas as pl
from jax.experimental.pallas import tpu as pltpu
from jax.experimental.pallas import tpu_sc as plsc

_N = 10000
_E = 320000
_DIN = 128
_H = 64
_ED = 16
_L = 3
_OUT = 2
_BN_EPS = 1e-5
_PN_EPS = 1e-5

_NC = 2
_NS = 16
_TILES = _NC * _NS
_EPT = _E // _TILES
_CH = 80
_NCH = _EPT // _CH
_RPT = 624
_RPT_LAST = _N - (_NS - 1) * _RPT


def _mm_bias_body(a_ref, w_ref, b_ref, o_ref):
    o_ref[...] = (
        jnp.dot(a_ref[...], w_ref[...], preferred_element_type=jnp.float32)
        + b_ref[...]
    )


def _mm_bias(a, w, b, block_rows):
    m, k = a.shape
    _, h = w.shape
    return pl.pallas_call(
        _mm_bias_body,
        grid=(m // block_rows,),
        in_specs=[
            pl.BlockSpec((block_rows, k), lambda i: (i, 0)),
            pl.BlockSpec((k, h), lambda i: (0, 0)),
            pl.BlockSpec((1, h), lambda i: (0, 0)),
        ],
        out_specs=pl.BlockSpec((block_rows, h), lambda i: (i, 0)),
        out_shape=jax.ShapeDtypeStruct((m, h), jnp.float32),
    )(a, w, b.reshape(1, h))


def _node_update_body(h_ref, p_ref, eps_ref, w1_ref, b1_ref, g1_ref, bt1_ref,
                      w2_ref, b2_ref, bg_ref, bb_ref, o_ref, *, mode):
    bn_scale = 1.0 / jnp.sqrt(1.0 + _BN_EPS)
    z = (1.0 + eps_ref[0, 0]) * h_ref[...] + p_ref[0] + p_ref[1]
    z = jnp.dot(z, w1_ref[...], preferred_element_type=jnp.float32) + b1_ref[...]
    z = z * (g1_ref[...] * bn_scale) + bt1_ref[...]
    z = jnp.maximum(z, 0.0)
    z = jnp.dot(z, w2_ref[...], preferred_element_type=jnp.float32) + b2_ref[...]
    z = z * (bg_ref[...] * bn_scale) + bb_ref[...]
    z = jnp.maximum(z, 0.0)
    if mode == "pairnorm":
        z = z - jnp.mean(z, axis=0, keepdims=True)
        denom = jnp.sqrt(_PN_EPS + jnp.mean(jnp.sum(z * z, axis=-1)))
        z = z / denom
    o_ref[...] = z


def _final_body(h_ref, p_ref, eps_ref, w1_ref, b1_ref, g1_ref, bt1_ref,
                w2_ref, b2_ref, bg_ref, bb_ref, wc_ref, bc_ref, o_ref):
    bn_scale = 1.0 / jnp.sqrt(1.0 + _BN_EPS)
    z = (1.0 + eps_ref[0, 0]) * h_ref[...] + p_ref[0] + p_ref[1]
    z = jnp.dot(z, w1_ref[...], preferred_element_type=jnp.float32) + b1_ref[...]
    z = z * (g1_ref[...] * bn_scale) + bt1_ref[...]
    z = jnp.maximum(z, 0.0)
    z = jnp.dot(z, w2_ref[...], preferred_element_type=jnp.float32) + b2_ref[...]
    z = z * (bg_ref[...] * bn_scale) + bb_ref[...]
    z = jnp.maximum(z, 0.0)
    o_ref[...] = (
        jnp.dot(z, wc_ref[...], preferred_element_type=jnp.float32) + bc_ref[...]
    )


def _node_update(h, parts, eps_l, w1, b1, g1, bt1, w2, b2, bg, bb):
    return pl.pallas_call(
        functools.partial(_node_update_body, mode="pairnorm"),
        out_shape=jax.ShapeDtypeStruct((_N, _H), jnp.float32),
    )(h, parts, eps_l.reshape(1, 1), w1, b1.reshape(1, _H), g1.reshape(1, _H),
      bt1.reshape(1, _H), w2, b2.reshape(1, _H), bg.reshape(1, _H),
      bb.reshape(1, _H))


def _node_final(h, parts, eps_l, w1, b1, g1, bt1, w2, b2, bg, bb, wc, bc):
    return pl.pallas_call(
        _final_body,
        out_shape=jax.ShapeDtypeStruct((_N, _OUT), jnp.float32),
    )(h, parts, eps_l.reshape(1, 1), w1, b1.reshape(1, _H), g1.reshape(1, _H),
      bt1.reshape(1, _H), w2, b2.reshape(1, _H), bg.reshape(1, _H),
      bb.reshape(1, _H), wc, bc.reshape(1, _OUT))


def _sc_agg_body(ei_hbm, e_hbm, h_hbm, z_hbm, out_hbm,
                 idx0, idx1, e0, e1, g0, g1, agg_sh,
                 is0, is1, es0, es1, gs0, gs1):
    c = lax.axis_index("c")
    s = lax.axis_index("s")
    wid = c * _NS + s

    row0 = pl.multiple_of(s * _RPT, 8)
    last = s == _NS - 1

    @pl.when(~last)
    def _zero_main():
        pltpu.sync_copy(z_hbm.at[pl.ds(0, _RPT)], agg_sh.at[pl.ds(row0, _RPT)])

    @pl.when(last)
    def _zero_last():
        pltpu.sync_copy(z_hbm, agg_sh.at[pl.ds((_NS - 1) * _RPT, _RPT_LAST)])

    plsc.subcore_barrier()

    base = wid * _EPT

    def _fetch(i, idx_v, e_v, i_sem, e_sem):
        off = pl.multiple_of(base + i * _CH, _CH)
        pltpu.make_async_copy(ei_hbm.at[:, pl.ds(off, _CH)], idx_v, i_sem).start()
        pltpu.make_async_copy(e_hbm.at[pl.ds(off, _CH)], e_v, e_sem).start()

    def _wait_idx(idx_v, i_sem):
        pltpu.make_async_copy(ei_hbm.at[:, pl.ds(0, _CH)], idx_v, i_sem).wait()

    def _gather(idx_v, g_v, g_sem):
        pltpu.make_async_copy(h_hbm.at[idx_v.at[0]], g_v, g_sem).start()

    def _compute_scatter(idx_v, e_v, g_v, e_sem, g_sem):
        pltpu.make_async_copy(e_hbm.at[pl.ds(0, _CH)], e_v, e_sem).wait()
        pltpu.make_async_copy(h_hbm.at[idx_v.at[0]], g_v, g_sem).wait()

        def row(r4, carry2):
            for u in range(4):
                r = r4 * 4 + u
                for j in range(_H // 16):
                    sl = pl.ds(j * 16, 16)
                    g_v[r, sl] = jnp.maximum(g_v[r, sl] + e_v[r, sl], 0.0)
            return carry2

        lax.fori_loop(0, _CH // 4, row, 0)
        pltpu.sync_copy(g_v, agg_sh.at[idx_v.at[1]], add=True)

    _fetch(0, idx0, e0, is0, es0)
    _fetch(1, idx1, e1, is1, es1)
    _wait_idx(idx0, is0)
    _gather(idx0, g0, gs0)

    def step(k, carry):
        i = k * 2
        _wait_idx(idx1, is1)
        _gather(idx1, g1, gs1)
        _compute_scatter(idx0, e0, g0, es0, gs0)
        _fetch(i + 2, idx0, e0, is0, es0)
        _wait_idx(idx0, is0)
        _gather(idx0, g0, gs0)
        _compute_scatter(idx1, e1, g1, es1, gs1)

        @pl.when(k < _NCH // 2 - 1)
        def _refill():
            _fetch(i + 3, idx1, e1, is1, es1)

        return carry

    lax.fori_loop(0, _NCH // 2, step, 0)
    _compute_scatter(idx0, e0, g0, es0, gs0)

    plsc.subcore_barrier()

    @pl.when(~last)
    def _read_main():
        pltpu.sync_copy(agg_sh.at[pl.ds(row0, _RPT)],
                        out_hbm.at[c, pl.ds(row0, _RPT)])

    @pl.when(last)
    def _read_last():
        pltpu.sync_copy(agg_sh.at[pl.ds((_NS - 1) * _RPT, _RPT_LAST)],
                        out_hbm.at[c, pl.ds((_NS - 1) * _RPT, _RPT_LAST)])


_sc_agg = functools.partial(
    pl.kernel,
    mesh=plsc.VectorSubcoreMesh(core_axis_name="c", subcore_axis_name="s"),
    out_type=jax.ShapeDtypeStruct((_NC, _N, _H), jnp.float32),
    scratch_types=[
        pltpu.VMEM((2, _CH), jnp.int32),
        pltpu.VMEM((2, _CH), jnp.int32),
        pltpu.VMEM((_CH, _H), jnp.float32),
        pltpu.VMEM((_CH, _H), jnp.float32),
        pltpu.VMEM((_CH, _H), jnp.float32),
        pltpu.VMEM((_CH, _H), jnp.float32),
        pltpu.VMEM_SHARED((_N, _H), jnp.float32),
        pltpu.SemaphoreType.DMA,
        pltpu.SemaphoreType.DMA,
        pltpu.SemaphoreType.DMA,
        pltpu.SemaphoreType.DMA,
        pltpu.SemaphoreType.DMA,
        pltpu.SemaphoreType.DMA,
    ],
    compiler_params=pltpu.CompilerParams(use_tc_tiling_on_sc=False),
)(_sc_agg_body)


def kernel(x, edge_index, edge_attr, W_in, b_in, We, be, eps, W1, b1, g1, bt1,
           W2, b2, bn_g, bn_b, Wc, bc):
    zrows = jnp.zeros((_RPT_LAST, _H), jnp.float32)

    h = _mm_bias(x, W_in, b_in, 2000)
    out = None
    for l in range(_L):
        e = _mm_bias(edge_attr, We[l], be[l], 8000)
        parts = _sc_agg(edge_index, e, h, zrows)
        if l < _L - 1:
            h = _node_update(h, parts, eps[l], W1[l], b1[l], g1[l], bt1[l],
                             W2[l], b2[l], bn_g[l], bn_b[l])
        else:
            out = _node_final(h, parts, eps[l], W1[l], b1[l], g1[l], bt1[l],
                              W2[l], b2[l], bn_g[l], bn_b[l], Wc, bc)
    return out


# speedup vs baseline: 7.2424x; 1.7778x over previous
import functools

import jax
import jax.numpy as jnp
from jax import lax
from jax.experimental import pallas as pl
from jax.experimental.pallas import tpu as pltpu
from jax.experimental.pallas import tpu_sc as plsc

_N = 10000
_E = 320000
_DIN = 128
_H = 64
_ED = 16
_L = 3
_OUT = 2
_BN_EPS = 1e-5
_PN_EPS = 1e-5

_NC = 2
_NS = 16
_TILES = _NC * _NS
_EPT = _E // _TILES
_CH = 80
_NCH = _EPT // _CH
_RPT = 624
_RPT_LAST = _N - (_NS - 1) * _RPT


def _mm_bias_body(a_ref, w_ref, b_ref, o_ref):
    o_ref[...] = (
        jnp.dot(a_ref[...], w_ref[...], preferred_element_type=jnp.float32)
        + b_ref[...]
    )


def _mm_bias(a, w, b, block_rows):
    m, k = a.shape
    _, h = w.shape
    return pl.pallas_call(
        _mm_bias_body,
        grid=(m // block_rows,),
        in_specs=[
            pl.BlockSpec((block_rows, k), lambda i: (i, 0)),
            pl.BlockSpec((k, h), lambda i: (0, 0)),
            pl.BlockSpec((1, h), lambda i: (0, 0)),
        ],
        out_specs=pl.BlockSpec((block_rows, h), lambda i: (i, 0)),
        out_shape=jax.ShapeDtypeStruct((m, h), jnp.float32),
    )(a, w, b.reshape(1, h))


def _node_update_body(h_ref, p_ref, eps_ref, w1_ref, b1_ref, g1_ref, bt1_ref,
                      w2_ref, b2_ref, bg_ref, bb_ref, o_ref, *, mode):
    bn_scale = 1.0 / jnp.sqrt(1.0 + _BN_EPS)
    p = p_ref[...]
    z = (1.0 + eps_ref[0, 0]) * h_ref[...] + p[:, :_H] + p[:, _H:]
    z = jnp.dot(z, w1_ref[...], preferred_element_type=jnp.float32) + b1_ref[...]
    z = z * (g1_ref[...] * bn_scale) + bt1_ref[...]
    z = jnp.maximum(z, 0.0)
    z = jnp.dot(z, w2_ref[...], preferred_element_type=jnp.float32) + b2_ref[...]
    z = z * (bg_ref[...] * bn_scale) + bb_ref[...]
    z = jnp.maximum(z, 0.0)
    if mode == "pairnorm":
        z = z - jnp.mean(z, axis=0, keepdims=True)
        denom = jnp.sqrt(_PN_EPS + jnp.mean(jnp.sum(z * z, axis=-1)))
        z = z / denom
    o_ref[...] = z


def _final_body(h_ref, p_ref, eps_ref, w1_ref, b1_ref, g1_ref, bt1_ref,
                w2_ref, b2_ref, bg_ref, bb_ref, wc_ref, bc_ref, o_ref):
    bn_scale = 1.0 / jnp.sqrt(1.0 + _BN_EPS)
    p = p_ref[...]
    z = (1.0 + eps_ref[0, 0]) * h_ref[...] + p[:, :_H] + p[:, _H:]
    z = jnp.dot(z, w1_ref[...], preferred_element_type=jnp.float32) + b1_ref[...]
    z = z * (g1_ref[...] * bn_scale) + bt1_ref[...]
    z = jnp.maximum(z, 0.0)
    z = jnp.dot(z, w2_ref[...], preferred_element_type=jnp.float32) + b2_ref[...]
    z = z * (bg_ref[...] * bn_scale) + bb_ref[...]
    z = jnp.maximum(z, 0.0)
    o_ref[...] = (
        jnp.dot(z, wc_ref[...], preferred_element_type=jnp.float32) + bc_ref[...]
    )


def _node_update(h, parts, eps_l, w1, b1, g1, bt1, w2, b2, bg, bb):
    return pl.pallas_call(
        functools.partial(_node_update_body, mode="pairnorm"),
        out_shape=jax.ShapeDtypeStruct((_N, _H), jnp.float32),
    )(h, parts, eps_l.reshape(1, 1), w1, b1.reshape(1, _H), g1.reshape(1, _H),
      bt1.reshape(1, _H), w2, b2.reshape(1, _H), bg.reshape(1, _H),
      bb.reshape(1, _H))


def _node_final(h, parts, eps_l, w1, b1, g1, bt1, w2, b2, bg, bb, wc, bc):
    return pl.pallas_call(
        _final_body,
        out_shape=jax.ShapeDtypeStruct((_N, _OUT), jnp.float32),
    )(h, parts, eps_l.reshape(1, 1), w1, b1.reshape(1, _H), g1.reshape(1, _H),
      bt1.reshape(1, _H), w2, b2.reshape(1, _H), bg.reshape(1, _H),
      bb.reshape(1, _H), wc, bc.reshape(1, _OUT))


def _sc_agg_body(src_hbm, dst_hbm, e_hbm, h_hbm, z_hbm, out_hbm,
                 ssrc, fdst, ebuf, gbuf, sdst, agg_sh,
                 isem, jsem, esem, gsem, csem):
    c = lax.axis_index("c")
    s = lax.axis_index("s")
    wid = c * _NS + s

    row0 = pl.multiple_of(s * _RPT, 8)
    last = s == _NS - 1

    @pl.when(~last)
    def _zero_main():
        pltpu.sync_copy(z_hbm.at[pl.ds(0, _RPT)], agg_sh.at[pl.ds(row0, _RPT)])

    @pl.when(last)
    def _zero_last():
        pltpu.sync_copy(z_hbm, agg_sh.at[pl.ds((_NS - 1) * _RPT, _RPT_LAST)])

    plsc.subcore_barrier()

    base = wid * _EPT

    def _fetch(i, a):
        off = pl.multiple_of(base + i * _CH, _CH)
        off2 = pl.multiple_of((base + i * _CH) // 2, _CH // 2)
        pltpu.make_async_copy(src_hbm.at[pl.ds(off, _CH)], ssrc[a], isem[a]).start()
        pltpu.make_async_copy(dst_hbm.at[pl.ds(off, _CH)], fdst[a], jsem[a]).start()
        pltpu.make_async_copy(e_hbm.at[pl.ds(off2, _CH // 2)], ebuf[a], esem[a]).start()

    def _gather(a):
        pltpu.make_async_copy(src_hbm.at[pl.ds(0, _CH)], ssrc[a], isem[a]).wait()
        pltpu.make_async_copy(h_hbm.at[ssrc[a]], gbuf[a], gsem[a]).start()

    def _wait_scat(a):
        pltpu.make_async_copy(gbuf[a], agg_sh.at[sdst[a]], csem[a]).wait()

    def _compute_scatter(a):
        pltpu.make_async_copy(e_hbm.at[pl.ds(0, _CH // 2)], ebuf[a], esem[a]).wait()
        pltpu.make_async_copy(h_hbm.at[ssrc[a]], gbuf[a], gsem[a]).wait()
        g_v = gbuf[a]
        e_v = ebuf[a]

        def row(kk, carry2):
            for u in range(4):
                r = kk * 4 + u
                er = 2 * kk + (u // 2)
                cb = (u % 2) * _H
                for j in range(_H // 16):
                    gsl = pl.ds(j * 16, 16)
                    esl = pl.ds(cb + j * 16, 16)
                    g_v[r, gsl] = jnp.maximum(g_v[r, gsl] + e_v[er, esl], 0.0)
            return carry2

        lax.fori_loop(0, _CH // 4, row, 0)
        pltpu.make_async_copy(dst_hbm.at[pl.ds(0, _CH)], fdst[a], jsem[a]).wait()
        for j in range(_CH // 16):
            sl = pl.ds(j * 16, 16)
            sdst[a][sl] = fdst[a][sl]
        pltpu.async_copy(g_v, agg_sh.at[sdst[a]], csem[a], add=True)

    for i in range(5):
        _fetch(i, i)
    for a in range(3):
        _gather(a)

    for i in (0, 1, 2):
        _gather((i + 3) % 6)
        _compute_scatter(i % 6)
        _fetch(i + 5, (i + 5) % 6)

    def step(k, carry):
        for u in range(6):
            i = 3 + 6 * k + u
            slot = (3 + u) % 6
            g_slot = (slot + 3) % 6

            @pl.when(i + 3 < _NCH)
            def _g():
                _wait_scat(g_slot)
                _gather(g_slot)

            _compute_scatter(slot)

            @pl.when(i + 5 < _NCH)
            def _f():
                _fetch(i + 5, (slot + 5) % 6)

        return carry

    lax.fori_loop(0, (_NCH - 5) // 6, step, 0)
    _compute_scatter(3)
    _compute_scatter(4)
    for a in range(6):
        _wait_scat(a)

    plsc.subcore_barrier()

    cm = pl.multiple_of(c * _H, _H)

    @pl.when(~last)
    def _read_main():
        pltpu.sync_copy(agg_sh.at[pl.ds(row0, _RPT)],
                        out_hbm.at[pl.ds(row0, _RPT), pl.ds(cm, _H)])

    @pl.when(last)
    def _read_last():
        pltpu.sync_copy(agg_sh.at[pl.ds((_NS - 1) * _RPT, _RPT_LAST)],
                        out_hbm.at[pl.ds((_NS - 1) * _RPT, _RPT_LAST),
                                   pl.ds(cm, _H)])


_sc_agg = functools.partial(
    pl.kernel,
    mesh=plsc.VectorSubcoreMesh(core_axis_name="c", subcore_axis_name="s"),
    out_type=jax.ShapeDtypeStruct((_N, 2 * _H), jnp.float32),
    scratch_types=[
        tuple(pltpu.VMEM((_CH,), jnp.int32) for _ in range(6)),
        tuple(pltpu.VMEM((_CH,), jnp.int32) for _ in range(6)),
        tuple(pltpu.VMEM((_CH // 2, 2 * _H), jnp.float32) for _ in range(6)),
        tuple(pltpu.VMEM((_CH, _H), jnp.float32) for _ in range(6)),
        tuple(pltpu.VMEM((_CH,), jnp.int32) for _ in range(6)),
        pltpu.VMEM_SHARED((_N, _H), jnp.float32),
        tuple(pltpu.SemaphoreType.DMA for _ in range(6)),
        tuple(pltpu.SemaphoreType.DMA for _ in range(6)),
        tuple(pltpu.SemaphoreType.DMA for _ in range(6)),
        tuple(pltpu.SemaphoreType.DMA for _ in range(6)),
        tuple(pltpu.SemaphoreType.DMA for _ in range(6)),
    ],
    compiler_params=pltpu.CompilerParams(use_tc_tiling_on_sc=False),
)(_sc_agg_body)


def kernel(x, edge_index, edge_attr, W_in, b_in, We, be, eps, W1, b1, g1, bt1,
           W2, b2, bn_g, bn_b, Wc, bc):
    src = edge_index[0]
    dst = edge_index[1]
    zrows = jnp.zeros((_RPT_LAST, _H), jnp.float32)

    a2 = edge_attr.reshape(_E // 2, 2 * _ED)
    we2 = jnp.zeros((_L, 2 * _ED, 2 * _H), jnp.float32)
    we2 = we2.at[:, :_ED, :_H].set(We).at[:, _ED:, _H:].set(We)
    be2 = jnp.concatenate([be, be], axis=-1)

    h = _mm_bias(x, W_in, b_in, 2000)
    out = None
    for l in range(_L):
        e2 = _mm_bias(a2, we2[l], be2[l], 8000)
        parts = _sc_agg(src, dst, e2, h, zrows)
        if l < _L - 1:
            h = _node_update(h, parts, eps[l], W1[l], b1[l], g1[l], bt1[l],
                             W2[l], b2[l], bn_g[l], bn_b[l])
        else:
            out = _node_final(h, parts, eps[l], W1[l], b1[l], g1[l], bt1[l],
                              W2[l], b2[l], bn_g[l], bn_b[l], Wc, bc)
    return out
